# baseline (speedup 1.0000x reference)
import functools

import jax
import jax.numpy as jnp
from jax import lax
from jax.experimental import pallas as pl
from jax.experimental.pallas import tpu as pltpu
from jax.experimental.pallas import tpu_sc as plsc

NC = 2
NS = 16
NW = NC * NS
L = 16

B = 8
C = 19
H = 512
W = 512
TH = 8
TW = 128

H_TC = 224
WPI = NW // B
HROWS = (H - H_TC) // WPI
P = TH * TW
NWB = W // TW
NHB = HROWS // TH
NCHUNK = NHB * NWB
NBUF = 4

TC_BH = H_TC // 2


def _sc_body(pred_hbm, y_hbm, out_hbm, xbuf, ybuf, accv, sem0, sem1, sem2, sem3):
    cid = lax.axis_index("c")
    sid = lax.axis_index("s")
    wid = sid * NC + cid
    b = wid // WPI
    h0 = H_TC + (wid % WPI) * HROWS
    sems = [sem0, sem1, sem2, sem3]

    def start(chunk, slot):
        h = h0 + (chunk // NWB) * TH
        wc = (chunk % NWB) * TW
        pltpu.async_copy(
            pred_hbm.at[b, :, pl.ds(h, TH), pl.ds(wc, TW)],
            xbuf.at[slot],
            sems[slot],
        )
        pltpu.async_copy(
            y_hbm.at[b, pl.ds(h, TH), pl.ds(wc, TW)], ybuf.at[slot], sems[slot]
        )

    def wait(slot):
        pltpu.make_async_copy(
            pred_hbm.at[0, :, pl.ds(0, TH), pl.ds(0, TW)],
            xbuf.at[slot],
            sems[slot],
        ).wait()
        pltpu.make_async_copy(
            y_hbm.at[0, pl.ds(0, TH), pl.ds(0, TW)], ybuf.at[slot], sems[slot]
        ).wait()

    iotav = lax.iota(jnp.int32, L)

    def compute(slot, acc):
        def rbody(r, acc):
            rvec = jnp.full((L,), 0, jnp.int32) + r
            for col in range(TW // L):
                o16 = col * L
                ys = ybuf[slot, r, pl.ds(o16, L)]
                lane = iotav + o16
                xs = [xbuf[slot, c, r, pl.ds(o16, L)] for c in range(C)]
                s = jnp.exp(xs[0])
                for c in range(1, C):
                    s = s + jnp.exp(xs[c])
                g = plsc.load_gather(xbuf.at[slot], [ys, rvec, lane])
                acc = acc + jnp.exp(g) / s
            return acc

        return lax.fori_loop(0, TH, rbody, acc)

    for slot in range(NBUF):
        start(slot, slot)

    def outer(t, acc):
        for slot in range(NBUF):
            chunk = t * NBUF + slot
            wait(slot)
            acc = compute(slot, acc)
            nxt = chunk + NBUF

            @pl.when(nxt < NCHUNK)
            def _():
                start(nxt, slot)
        return acc

    acc = lax.fori_loop(0, NCHUNK // NBUF, outer, jnp.zeros((L,), jnp.float32))
    accv[...] = acc
    pltpu.sync_copy(accv, out_hbm.at[wid])


def _nce_sc(pred, y2):
    mesh = plsc.VectorSubcoreMesh(
        core_axis_name="c", subcore_axis_name="s", num_cores=NC, num_subcores=NS
    )
    k = functools.partial(
        pl.kernel,
        out_type=jax.ShapeDtypeStruct((NW, L), jnp.float32),
        mesh=mesh,
        scratch_types=[
            pltpu.VMEM((NBUF, C, TH, TW), jnp.float32),
            pltpu.VMEM((NBUF, TH, TW), jnp.int32),
            pltpu.VMEM((L,), jnp.float32),
            pltpu.SemaphoreType.DMA,
            pltpu.SemaphoreType.DMA,
            pltpu.SemaphoreType.DMA,
            pltpu.SemaphoreType.DMA,
        ],
        compiler_params=pltpu.CompilerParams(
            use_tc_tiling_on_sc=True, needs_layout_passes=False
        ),
    )(_sc_body)
    return k(pred, y2)


def _tc_block(pred_ref, y_ref, out_ref):
    i = pl.program_id(0)
    j = pl.program_id(1)

    x = pred_ref[0]
    y = y_ref[0]
    c, bh, w = x.shape

    m = jnp.max(x, axis=0)
    e = jnp.exp(x - m[None])
    s = jnp.sum(e, axis=0)
    cls = jax.lax.broadcasted_iota(jnp.int32, (c, bh, w), 0)
    sel = jnp.sum(jnp.where(cls == y[None], e, 0.0), axis=0)
    partial = jnp.sum(sel / s)

    @pl.when(jnp.logical_and(i == 0, j == 0))
    def _():
        out_ref[0, 0] = 0.0

    out_ref[0, 0] += partial


def _nce_tc(pred, y2):
    grid = (B, H_TC // TC_BH)
    out = pl.pallas_call(
        _tc_block,
        grid=grid,
        in_specs=[
            pl.BlockSpec((1, C, TC_BH, W), lambda i, j: (i, 0, j, 0)),
            pl.BlockSpec((1, TC_BH, W), lambda i, j: (i, j, 0)),
        ],
        out_specs=pl.BlockSpec(
            (1, 1), lambda i, j: (0, 0), memory_space=pltpu.SMEM
        ),
        out_shape=jax.ShapeDtypeStruct((1, 1), jnp.float32),
    )(pred, y2)
    return out[0, 0]


def kernel(pred, y_true):
    b, c, h, w = pred.shape
    y2 = y_true.astype(jnp.int32)
    sc_part = _nce_sc(pred, y2)
    tc_part = _nce_tc(pred, y2)
    return (jnp.sum(sc_part) + tc_part) / jnp.float32(b * h * w)


# speedup vs baseline: 1.5416x; 1.5416x over previous
import functools

import jax
import jax.numpy as jnp
from jax import lax
from jax.experimental import pallas as pl
from jax.experimental.pallas import tpu as pltpu
from jax.experimental.pallas import tpu_sc as plsc

NC = 2
NS = 16
NW = NC * NS
L = 16

B = 8
C = 19
H = 512
W = 512
TH = 8
TW = 128

H_TC = 256
WPI = NW // B
HROWS = (H - H_TC) // WPI
P = TH * TW
NWB = W // TW
NHB = HROWS // TH
NCHUNK = NHB * NWB
NBUF = 4

TC_BH = H_TC // 2


def _sc_body(pred_hbm, y_hbm, out_hbm, xbuf, ybuf, accv, sem0, sem1, sem2, sem3):
    cid = lax.axis_index("c")
    sid = lax.axis_index("s")
    wid = sid * NC + cid
    b = wid // WPI
    h0 = H_TC + (wid % WPI) * HROWS
    sems = [sem0, sem1, sem2, sem3]

    def start(chunk, slot):
        h = h0 + (chunk // NWB) * TH
        wc = (chunk % NWB) * TW
        pltpu.async_copy(
            pred_hbm.at[b, :, pl.ds(h, TH), pl.ds(wc, TW)],
            xbuf.at[slot],
            sems[slot],
        )
        pltpu.async_copy(
            y_hbm.at[b, pl.ds(h, TH), pl.ds(wc, TW)], ybuf.at[slot], sems[slot]
        )

    def wait(slot):
        pltpu.make_async_copy(
            pred_hbm.at[0, :, pl.ds(0, TH), pl.ds(0, TW)],
            xbuf.at[slot],
            sems[slot],
        ).wait()
        pltpu.make_async_copy(
            y_hbm.at[0, pl.ds(0, TH), pl.ds(0, TW)], ybuf.at[slot], sems[slot]
        ).wait()

    iotav = lax.iota(jnp.int32, L)

    def compute(slot, acc):
        def jbody(j, acc):
            r = j // TH
            o16 = (j % TH) * L
            ys = ybuf[slot, r, pl.ds(o16, L)]
            lane = iotav + o16
            rvec = jnp.full((L,), 0, jnp.int32) + r
            xs = [xbuf[slot, c, r, pl.ds(o16, L)] for c in range(C)]
            s = jnp.exp(xs[0])
            for c in range(1, C):
                s = s + jnp.exp(xs[c])
            g = plsc.load_gather(xbuf.at[slot], [ys, rvec, lane])
            return acc + jnp.exp(g) / s

        return lax.fori_loop(0, P // L, jbody, acc)

    for slot in range(NBUF):
        start(slot, slot)

    def outer(t, acc):
        for slot in range(NBUF):
            chunk = t * NBUF + slot
            wait(slot)
            acc = compute(slot, acc)
            nxt = chunk + NBUF

            @pl.when(nxt < NCHUNK)
            def _():
                start(nxt, slot)
        return acc

    acc = lax.fori_loop(0, NCHUNK // NBUF, outer, jnp.zeros((L,), jnp.float32))
    accv[...] = acc
    pltpu.sync_copy(accv, out_hbm.at[wid])


def _nce_sc(pred, y2):
    mesh = plsc.VectorSubcoreMesh(
        core_axis_name="c", subcore_axis_name="s", num_cores=NC, num_subcores=NS
    )
    k = functools.partial(
        pl.kernel,
        out_type=jax.ShapeDtypeStruct((NW, L), jnp.float32),
        mesh=mesh,
        scratch_types=[
            pltpu.VMEM((NBUF, C, TH, TW), jnp.float32),
            pltpu.VMEM((NBUF, TH, TW), jnp.int32),
            pltpu.VMEM((L,), jnp.float32),
            pltpu.SemaphoreType.DMA,
            pltpu.SemaphoreType.DMA,
            pltpu.SemaphoreType.DMA,
            pltpu.SemaphoreType.DMA,
        ],
        compiler_params=pltpu.CompilerParams(
            use_tc_tiling_on_sc=True, needs_layout_passes=False
        ),
    )(_sc_body)
    return k(pred, y2)


def _tc_block(pred_ref, y_ref, out_ref):
    i = pl.program_id(0)
    j = pl.program_id(1)

    x = pred_ref[0]
    y = y_ref[0]
    c, bh, w = x.shape

    m = jnp.max(x, axis=0)
    e = jnp.exp(x - m[None])
    s = jnp.sum(e, axis=0)
    cls = jax.lax.broadcasted_iota(jnp.int32, (c, bh, w), 0)
    sel = jnp.sum(jnp.where(cls == y[None], e, 0.0), axis=0)
    partial = jnp.sum(sel / s)

    @pl.when(jnp.logical_and(i == 0, j == 0))
    def _():
        out_ref[0, 0] = 0.0

    out_ref[0, 0] += partial


def _nce_tc(pred, y2):
    grid = (B, H_TC // TC_BH)
    out = pl.pallas_call(
        _tc_block,
        grid=grid,
        in_specs=[
            pl.BlockSpec((1, C, TC_BH, W), lambda i, j: (i, 0, j, 0)),
            pl.BlockSpec((1, TC_BH, W), lambda i, j: (i, j, 0)),
        ],
        out_specs=pl.BlockSpec(
            (1, 1), lambda i, j: (0, 0), memory_space=pltpu.SMEM
        ),
        out_shape=jax.ShapeDtypeStruct((1, 1), jnp.float32),
    )(pred, y2)
    return out[0, 0]


def kernel(pred, y_true):
    b, c, h, w = pred.shape
    y2 = y_true.astype(jnp.int32)
    sc_part = _nce_sc(pred, y2)
    tc_part = _nce_tc(pred, y2)
    return (jnp.sum(sc_part) + tc_part) / jnp.float32(b * h * w)


# speedup vs baseline: 1.5595x; 1.0116x over previous
import functools

import jax
import jax.numpy as jnp
from jax import lax
from jax.experimental import pallas as pl
from jax.experimental.pallas import tpu as pltpu
from jax.experimental.pallas import tpu_sc as plsc

NC = 2
NS = 16
NW = NC * NS
L = 16

B = 8
C = 19
H = 512
W = 512
TH = 8
TW = 128

H_TC = 256
WPI = NW // B
HROWS = (H - H_TC) // WPI
P = TH * TW
NWB = W // TW
NHB = HROWS // TH
NCHUNK = NHB * NWB
NBUF = 4

TC_BH = 256


def _sc_body(pred_hbm, y_hbm, out_hbm, xbuf, ybuf, accv, sem0, sem1, sem2, sem3):
    cid = lax.axis_index("c")
    sid = lax.axis_index("s")
    wid = sid * NC + cid
    b = wid // WPI
    h0 = H_TC + (wid % WPI) * HROWS
    sems = [sem0, sem1, sem2, sem3]

    def start(chunk, slot):
        h = h0 + (chunk // NWB) * TH
        wc = (chunk % NWB) * TW
        pltpu.async_copy(
            pred_hbm.at[b, :, pl.ds(h, TH), pl.ds(wc, TW)],
            xbuf.at[slot],
            sems[slot],
        )
        pltpu.async_copy(
            y_hbm.at[b, pl.ds(h, TH), pl.ds(wc, TW)], ybuf.at[slot], sems[slot]
        )

    def wait(slot):
        pltpu.make_async_copy(
            pred_hbm.at[0, :, pl.ds(0, TH), pl.ds(0, TW)],
            xbuf.at[slot],
            sems[slot],
        ).wait()
        pltpu.make_async_copy(
            y_hbm.at[0, pl.ds(0, TH), pl.ds(0, TW)], ybuf.at[slot], sems[slot]
        ).wait()

    iotav = lax.iota(jnp.int32, L)

    def compute(slot, acc):
        def jbody(j, acc):
            r = j // TH
            o16 = (j % TH) * L
            ys = ybuf[slot, r, pl.ds(o16, L)]
            lane = iotav + o16
            rvec = jnp.full((L,), 0, jnp.int32) + r
            xs = [xbuf[slot, c, r, pl.ds(o16, L)] for c in range(C)]
            s = jnp.exp(xs[0])
            for c in range(1, C):
                s = s + jnp.exp(xs[c])
            g = plsc.load_gather(xbuf.at[slot], [ys, rvec, lane])
            return acc + jnp.exp(g) / s

        return lax.fori_loop(0, P // L, jbody, acc)

    for slot in range(NBUF):
        start(slot, slot)

    def outer(t, acc):
        for slot in range(NBUF):
            chunk = t * NBUF + slot
            wait(slot)
            acc = compute(slot, acc)
            nxt = chunk + NBUF

            @pl.when(nxt < NCHUNK)
            def _():
                start(nxt, slot)
        return acc

    acc = lax.fori_loop(0, NCHUNK // NBUF, outer, jnp.zeros((L,), jnp.float32))
    accv[...] = acc
    pltpu.sync_copy(accv, out_hbm.at[wid])


def _nce_sc(pred, y2):
    mesh = plsc.VectorSubcoreMesh(
        core_axis_name="c", subcore_axis_name="s", num_cores=NC, num_subcores=NS
    )
    k = functools.partial(
        pl.kernel,
        out_type=jax.ShapeDtypeStruct((NW, L), jnp.float32),
        mesh=mesh,
        scratch_types=[
            pltpu.VMEM((NBUF, C, TH, TW), jnp.float32),
            pltpu.VMEM((NBUF, TH, TW), jnp.int32),
            pltpu.VMEM((L,), jnp.float32),
            pltpu.SemaphoreType.DMA,
            pltpu.SemaphoreType.DMA,
            pltpu.SemaphoreType.DMA,
            pltpu.SemaphoreType.DMA,
        ],
        compiler_params=pltpu.CompilerParams(
            use_tc_tiling_on_sc=True, needs_layout_passes=False
        ),
    )(_sc_body)
    return k(pred, y2)


def _tc_block(pred_ref, y_ref, out_ref):
    i = pl.program_id(0)
    j = pl.program_id(1)

    x = pred_ref[0]
    y = y_ref[0]
    c, bh, w = x.shape

    m = jnp.max(x, axis=0)
    e = jnp.exp(x - m[None])
    s = jnp.sum(e, axis=0)
    cls = jax.lax.broadcasted_iota(jnp.int32, (c, bh, w), 0)
    sel = jnp.sum(jnp.where(cls == y[None], e, 0.0), axis=0)
    partial = jnp.sum(sel / s)

    @pl.when(jnp.logical_and(i == 0, j == 0))
    def _():
        out_ref[0, 0] = 0.0

    out_ref[0, 0] += partial


def _nce_tc(pred, y2):
    grid = (B, H_TC // TC_BH)
    out = pl.pallas_call(
        _tc_block,
        grid=grid,
        in_specs=[
            pl.BlockSpec((1, C, TC_BH, W), lambda i, j: (i, 0, j, 0)),
            pl.BlockSpec((1, TC_BH, W), lambda i, j: (i, j, 0)),
        ],
        out_specs=pl.BlockSpec(
            (1, 1), lambda i, j: (0, 0), memory_space=pltpu.SMEM
        ),
        out_shape=jax.ShapeDtypeStruct((1, 1), jnp.float32),
    )(pred, y2)
    return out[0, 0]


def kernel(pred, y_true):
    b, c, h, w = pred.shape
    y2 = y_true.astype(jnp.int32)
    sc_part = _nce_sc(pred, y2)
    tc_part = _nce_tc(pred, y2)
    return (jnp.sum(sc_part) + tc_part) / jnp.float32(b * h * w)


# speedup vs baseline: 1.5821x; 1.0145x over previous
import functools

import jax
import jax.numpy as jnp
from jax import lax
from jax.experimental import pallas as pl
from jax.experimental.pallas import tpu as pltpu
from jax.experimental.pallas import tpu_sc as plsc

NC = 2
NS = 16
NW = NC * NS
L = 16

B = 8
C = 19
H = 512
W = 512
TH = 8
TW = 128

H_TC = 256
WPI = NW // B
HROWS = (H - H_TC) // WPI
P = TH * TW
NWB = W // TW
NHB = HROWS // TH
NCHUNK = NHB * NWB
NBUF = 6

TC_BH = 256


def _sc_body(pred_hbm, y_hbm, out_hbm, xbuf, ybuf, accv, sem0, sem1, sem2, sem3, sem4, sem5):
    cid = lax.axis_index("c")
    sid = lax.axis_index("s")
    wid = sid * NC + cid
    b = wid // WPI
    h0 = H_TC + (wid % WPI) * HROWS
    sems = [sem0, sem1, sem2, sem3, sem4, sem5]

    def start(chunk, slot):
        h = h0 + (chunk // NWB) * TH
        wc = (chunk % NWB) * TW
        pltpu.async_copy(
            pred_hbm.at[b, :, pl.ds(h, TH), pl.ds(wc, TW)],
            xbuf.at[slot],
            sems[slot],
        )
        pltpu.async_copy(
            y_hbm.at[b, pl.ds(h, TH), pl.ds(wc, TW)], ybuf.at[slot], sems[slot]
        )

    def wait(slot):
        pltpu.make_async_copy(
            pred_hbm.at[0, :, pl.ds(0, TH), pl.ds(0, TW)],
            xbuf.at[slot],
            sems[slot],
        ).wait()
        pltpu.make_async_copy(
            y_hbm.at[0, pl.ds(0, TH), pl.ds(0, TW)], ybuf.at[slot], sems[slot]
        ).wait()

    iotav = lax.iota(jnp.int32, L)

    def compute(slot, acc):
        def jbody(j, acc):
            r = j // TH
            o16 = (j % TH) * L
            ys = ybuf[slot, r, pl.ds(o16, L)]
            lane = iotav + o16
            rvec = jnp.full((L,), 0, jnp.int32) + r
            xs = [xbuf[slot, c, r, pl.ds(o16, L)] for c in range(C)]
            s = jnp.exp(xs[0])
            for c in range(1, C):
                s = s + jnp.exp(xs[c])
            g = plsc.load_gather(xbuf.at[slot], [ys, rvec, lane])
            return acc + jnp.exp(g) / s

        return lax.fori_loop(0, P // L, jbody, acc)

    for slot in range(NBUF):
        start(slot, slot)

    def outer(t, acc):
        for slot in range(NBUF):
            chunk = t * NBUF + slot
            wait(slot)
            acc = compute(slot, acc)
            nxt = chunk + NBUF

            @pl.when(nxt < NCHUNK)
            def _():
                start(nxt, slot)
        return acc

    acc = lax.fori_loop(0, NCHUNK // NBUF, outer, jnp.zeros((L,), jnp.float32))
    accv[...] = acc
    pltpu.sync_copy(accv, out_hbm.at[wid])


def _nce_sc(pred, y2):
    mesh = plsc.VectorSubcoreMesh(
        core_axis_name="c", subcore_axis_name="s", num_cores=NC, num_subcores=NS
    )
    k = functools.partial(
        pl.kernel,
        out_type=jax.ShapeDtypeStruct((NW, L), jnp.float32),
        mesh=mesh,
        scratch_types=[
            pltpu.VMEM((NBUF, C, TH, TW), jnp.float32),
            pltpu.VMEM((NBUF, TH, TW), jnp.int32),
            pltpu.VMEM((L,), jnp.float32),
            pltpu.SemaphoreType.DMA,
            pltpu.SemaphoreType.DMA,
            pltpu.SemaphoreType.DMA,
            pltpu.SemaphoreType.DMA,
            pltpu.SemaphoreType.DMA,
            pltpu.SemaphoreType.DMA,
        ],
        compiler_params=pltpu.CompilerParams(
            use_tc_tiling_on_sc=True, needs_layout_passes=False
        ),
    )(_sc_body)
    return k(pred, y2)


def _tc_block(pred_ref, y_ref, out_ref):
    i = pl.program_id(0)
    j = pl.program_id(1)

    x = pred_ref[0]
    y = y_ref[0]
    c, bh, w = x.shape

    m = jnp.max(x, axis=0)
    e = jnp.exp(x - m[None])
    s = jnp.sum(e, axis=0)
    cls = jax.lax.broadcasted_iota(jnp.int32, (c, bh, w), 0)
    sel = jnp.sum(jnp.where(cls == y[None], e, 0.0), axis=0)
    partial = jnp.sum(sel / s)

    @pl.when(jnp.logical_and(i == 0, j == 0))
    def _():
        out_ref[0, 0] = 0.0

    out_ref[0, 0] += partial


def _nce_tc(pred, y2):
    grid = (B, H_TC // TC_BH)
    out = pl.pallas_call(
        _tc_block,
        grid=grid,
        in_specs=[
            pl.BlockSpec((1, C, TC_BH, W), lambda i, j: (i, 0, j, 0)),
            pl.BlockSpec((1, TC_BH, W), lambda i, j: (i, j, 0)),
        ],
        out_specs=pl.BlockSpec(
            (1, 1), lambda i, j: (0, 0), memory_space=pltpu.SMEM
        ),
        out_shape=jax.ShapeDtypeStruct((1, 1), jnp.float32),
    )(pred, y2)
    return out[0, 0]


def kernel(pred, y_true):
    b, c, h, w = pred.shape
    y2 = y_true.astype(jnp.int32)
    sc_part = _nce_sc(pred, y2)
    tc_part = _nce_tc(pred, y2)
    return (jnp.sum(sc_part) + tc_part) / jnp.float32(b * h * w)
